# baseline (speedup 1.0000x reference)
import functools

import jax
import jax.numpy as jnp
from jax.experimental import pallas as pl
from jax.experimental.pallas import tpu as pltpu

_EPS = 1e-5
_F32 = jnp.float32


def _l0_kernel(x_ref, m_ref, w_ref, xm_ref, s1_ref, s2_ref, cnt_ref):
    t = pl.program_id(0)
    z = jnp.dot(x_ref[...], w_ref[...], preferred_element_type=_F32)
    m = m_ref[...]
    xm = z * m
    xm_ref[...] = xm
    s1 = jnp.sum(xm, axis=0, keepdims=True)
    s2 = jnp.sum(xm * xm, axis=0, keepdims=True)
    c = jnp.sum(m, axis=0, keepdims=True)

    @pl.when(t == 0)
    def _():
        s1_ref[...] = s1
        s2_ref[...] = s2
        cnt_ref[...] = c

    @pl.when(t != 0)
    def _():
        s1_ref[...] += s1
        s2_ref[...] += s2
        cnt_ref[...] += c


def _norm_kernel(x_ref, m_ref, g_ref, b_ref, s1_ref, s2_ref, cnt_ref, y_ref):
    c = cnt_ref[0, 0] + 1e-8
    mean = s1_ref[...] / c
    var = s2_ref[...] / c - mean * mean
    inv = jax.lax.rsqrt(var + _EPS) * g_ref[...]
    y = (x_ref[...] - mean) * inv + b_ref[...]
    y_ref[...] = jnp.maximum(y, 0.0) * m_ref[...]


def _conv_bn_kernel(x_ref, w_ref, m_ref, g_ref, b_ref, y_ref, *, n_taps):
    t = pl.program_id(0)
    z = jnp.dot(x_ref[0], w_ref[0], preferred_element_type=_F32)

    @pl.when(t == 0)
    def _():
        y_ref[...] = z

    @pl.when(t != 0)
    def _():
        y_ref[...] += z

    @pl.when(t == n_taps - 1)
    def _():
        m = m_ref[...]
        xm = y_ref[...] * m
        c = jnp.sum(m) + 1e-8
        s1 = jnp.sum(xm, axis=0, keepdims=True)
        s2 = jnp.sum(xm * xm, axis=0, keepdims=True)
        mean = s1 / c
        var = s2 / c - mean * mean
        inv = jax.lax.rsqrt(var + _EPS) * g_ref[...]
        y = jnp.maximum((xm - mean) * inv + b_ref[...], 0.0) * m
        y_ref[...] = y


def _conv_bn_dec_kernel(x_ref, w_ref, m_ref, g_ref, b_ref, dw1_ref, db1_ref,
                        dw2_ref, db2_ref, p_ref, acc_ref, *, n_taps):
    t = pl.program_id(0)
    z = jnp.dot(x_ref[0], w_ref[0], preferred_element_type=_F32)

    @pl.when(t == 0)
    def _():
        acc_ref[...] = z

    @pl.when(t != 0)
    def _():
        acc_ref[...] += z

    @pl.when(t == n_taps - 1)
    def _():
        m = m_ref[...]
        xm = acc_ref[...] * m
        c = jnp.sum(m) + 1e-8
        s1 = jnp.sum(xm, axis=0, keepdims=True)
        s2 = jnp.sum(xm * xm, axis=0, keepdims=True)
        mean = s1 / c
        var = s2 / c - mean * mean
        inv = jax.lax.rsqrt(var + _EPS) * g_ref[...]
        y = jnp.maximum((xm - mean) * inv + b_ref[...], 0.0) * m
        h = jnp.maximum(
            jnp.dot(y, dw1_ref[...], preferred_element_type=_F32)
            + db1_ref[...], 0.0)
        p = jnp.dot(h, dw2_ref[...], preferred_element_type=_F32) + db2_ref[...]
        p_ref[...] = p * m


def _pool2(m):
    s = m.shape[1] // 2
    return m.reshape(m.shape[0], s, 2, s, 2, s, 2).max(axis=(2, 4, 6))


def _taps_stride2(y, cin):
    o = y.shape[1] // 2
    yp = jnp.pad(y, ((0, 0), (1, 1), (1, 1), (1, 1), (0, 0)))
    cols = [
        yp[:, a:a + 2 * o:2, b:b + 2 * o:2, c:c + 2 * o:2, :].reshape(
            2 * o ** 3, cin)
        for a in range(3) for b in range(3) for c in range(3)
    ]
    return jnp.stack(cols, axis=0)


def _conv_layer(taps, w, mrow, g, b, cout):
    n_taps, r, cin = taps.shape
    return pl.pallas_call(
        functools.partial(_conv_bn_kernel, n_taps=n_taps),
        grid=(n_taps,),
        in_specs=[
            pl.BlockSpec((1, r, cin), lambda t: (t, 0, 0)),
            pl.BlockSpec((1, cin, cout), lambda t: (t, 0, 0)),
            pl.BlockSpec((r, 1), lambda t: (0, 0)),
            pl.BlockSpec((1, cout), lambda t: (0, 0)),
            pl.BlockSpec((1, cout), lambda t: (0, 0)),
        ],
        out_specs=pl.BlockSpec((r, cout), lambda t: (0, 0)),
        out_shape=jax.ShapeDtypeStruct((r, cout), _F32),
        compiler_params=pltpu.CompilerParams(
            vmem_limit_bytes=64 * 1024 * 1024),
    )(taps, w, mrow, g, b)


def _conv_layer_dec(taps, w, mrow, g, b, dw1, db1, dw2, db2, cout):
    n_taps, r, cin = taps.shape
    hid = dw1.shape[1]
    return pl.pallas_call(
        functools.partial(_conv_bn_dec_kernel, n_taps=n_taps),
        grid=(n_taps,),
        in_specs=[
            pl.BlockSpec((1, r, cin), lambda t: (t, 0, 0)),
            pl.BlockSpec((1, cin, cout), lambda t: (t, 0, 0)),
            pl.BlockSpec((r, 1), lambda t: (0, 0)),
            pl.BlockSpec((1, cout), lambda t: (0, 0)),
            pl.BlockSpec((1, cout), lambda t: (0, 0)),
            pl.BlockSpec((cout, hid), lambda t: (0, 0)),
            pl.BlockSpec((1, hid), lambda t: (0, 0)),
            pl.BlockSpec((hid, 1), lambda t: (0, 0)),
            pl.BlockSpec((1, 1), lambda t: (0, 0)),
        ],
        out_specs=pl.BlockSpec((r, 1), lambda t: (0, 0)),
        out_shape=jax.ShapeDtypeStruct((r, 1), _F32),
        scratch_shapes=[pltpu.VMEM((r, cout), _F32)],
    )(taps, w, mrow, g, b, dw1, db1, dw2, db2)


def kernel(images, mask, conv_w0, bn_g0, bn_b0, conv_w1, bn_g1, bn_b1,
           conv_w2, bn_g2, bn_b2, conv_w3, bn_g3, bn_b3,
           dec_w1, dec_b1, dec_w2, dec_b2):
    B, _, D = images.shape[0], images.shape[1], images.shape[2]
    m0 = mask.astype(_F32)[:, 0]
    x = images[:, 0] * m0

    m1 = _pool2(m0)
    m2 = _pool2(m1)
    m3 = _pool2(m2)

    r0 = B * D ** 3
    xp = jnp.pad(x, ((0, 0), (1, 1), (1, 1), (1, 1)))
    cols = [
        xp[:, a:a + D, b:b + D, c:c + D]
        for a in range(3) for b in range(3) for c in range(3)
    ]
    x0 = jnp.stack(cols, axis=-1).reshape(r0, 27)
    w0 = conv_w0.reshape(27, 64)
    m0r = m0.reshape(r0, 1)
    blk = r0 // 16
    xm0, s1, s2, cnt = pl.pallas_call(
        _l0_kernel,
        grid=(16,),
        in_specs=[
            pl.BlockSpec((blk, 27), lambda i: (i, 0)),
            pl.BlockSpec((blk, 1), lambda i: (i, 0)),
            pl.BlockSpec((27, 64), lambda i: (0, 0)),
        ],
        out_specs=[
            pl.BlockSpec((blk, 64), lambda i: (i, 0)),
            pl.BlockSpec((1, 64), lambda i: (0, 0)),
            pl.BlockSpec((1, 64), lambda i: (0, 0)),
            pl.BlockSpec((1, 1), lambda i: (0, 0)),
        ],
        out_shape=[
            jax.ShapeDtypeStruct((r0, 64), _F32),
            jax.ShapeDtypeStruct((1, 64), _F32),
            jax.ShapeDtypeStruct((1, 64), _F32),
            jax.ShapeDtypeStruct((1, 1), _F32),
        ],
    )(x0, m0r, w0)

    y0 = pl.pallas_call(
        _norm_kernel,
        grid=(16,),
        in_specs=[
            pl.BlockSpec((blk, 64), lambda i: (i, 0)),
            pl.BlockSpec((blk, 1), lambda i: (i, 0)),
            pl.BlockSpec((1, 64), lambda i: (0, 0)),
            pl.BlockSpec((1, 64), lambda i: (0, 0)),
            pl.BlockSpec((1, 64), lambda i: (0, 0)),
            pl.BlockSpec((1, 64), lambda i: (0, 0)),
            pl.BlockSpec((1, 1), lambda i: (0, 0)),
        ],
        out_specs=pl.BlockSpec((blk, 64), lambda i: (i, 0)),
        out_shape=jax.ShapeDtypeStruct((r0, 64), _F32),
    )(xm0, m0r, bn_g0.reshape(1, 64), bn_b0.reshape(1, 64), s1, s2, cnt)

    y = y0
    g = D
    chans = [64, 128, 256]
    convs = [(conv_w1, bn_g1, bn_b1), (conv_w2, bn_g2, bn_b2)]
    masks = [m1, m2]
    for i, (cw, bg, bb) in enumerate(convs):
        cin, cout = chans[i], chans[i + 1]
        taps = _taps_stride2(y.reshape(B, g, g, g, cin), cin)
        g = g // 2
        r = B * g ** 3
        y = _conv_layer(taps, cw.reshape(27, cin, cout),
                        masks[i].reshape(r, 1), bg.reshape(1, cout),
                        bb.reshape(1, cout), cout)

    cin, cout = 256, 768
    taps = _taps_stride2(y.reshape(B, g, g, g, cin), cin)
    g = g // 2
    r = B * g ** 3
    p = _conv_layer_dec(taps, conv_w3.reshape(27, cin, cout),
                        m3.reshape(r, 1), bn_g3.reshape(1, cout),
                        bn_b3.reshape(1, cout), dec_w1.T,
                        dec_b1.reshape(1, -1), dec_w2.T,
                        dec_b2.reshape(1, 1), cout)

    recon = jnp.zeros((B, 1, D, D, D), _F32)
    recon = recon.at[:, :, :g, :g, :g].set(p.reshape(B, 1, g, g, g))
    return recon


# speedup vs baseline: 10.5089x; 10.5089x over previous
import functools

import jax
import jax.numpy as jnp
from jax.experimental import pallas as pl
from jax.experimental.pallas import tpu as pltpu

_EPS = 1e-5
_F32 = jnp.float32


def _expand_mask4(x, blk):
    m4 = x[:, 54:58]
    return jnp.concatenate(
        [jnp.broadcast_to(m4[:, j:j + 1], (blk, 64)) for j in range(4)],
        axis=1)


def _l0_stats_kernel(x_ref, w_ref, s1_ref, s2_ref, cnt_ref):
    t = pl.program_id(0)
    x = x_ref[...]
    z = jnp.dot(x, w_ref[...], preferred_element_type=_F32)
    m256 = _expand_mask4(x, x.shape[0])
    xm = z * m256
    s1 = jnp.sum(xm, axis=0, keepdims=True)
    s2 = jnp.sum(xm * xm, axis=0, keepdims=True)
    c = jnp.sum(x[:, 54:58])[None, None]

    @pl.when(t == 0)
    def _():
        s1_ref[...] = s1
        s2_ref[...] = s2
        cnt_ref[...] = c

    @pl.when(t != 0)
    def _():
        s1_ref[...] += s1
        s2_ref[...] += s2
        cnt_ref[...] += c


def _fold4(v):
    return v[:, 0:64] + v[:, 64:128] + v[:, 128:192] + v[:, 192:256]


def _l0_norm_kernel(x_ref, w_ref, g_ref, b_ref, s1_ref, s2_ref, cnt_ref,
                    y_ref):
    c = cnt_ref[0, 0] + 1e-8
    mean = _fold4(s1_ref[...]) / c
    var = _fold4(s2_ref[...]) / c - mean * mean
    inv = jax.lax.rsqrt(var + _EPS) * g_ref[...]
    mean4 = jnp.concatenate([mean] * 4, axis=1)
    inv4 = jnp.concatenate([inv] * 4, axis=1)
    b4 = jnp.concatenate([b_ref[...]] * 4, axis=1)
    x = x_ref[...]
    z = jnp.dot(x, w_ref[...], preferred_element_type=_F32)
    m256 = _expand_mask4(x, x.shape[0])
    y = (z * m256 - mean4) * inv4 + b4
    y_ref[...] = jnp.maximum(y, 0.0) * m256


def _conv_bn_kernel(x_ref, w_ref, g_ref, b_ref, y_ref, *, n_grp, mcol):
    t = pl.program_id(0)
    x = x_ref[0]
    z = jnp.dot(x, w_ref[0], preferred_element_type=_F32)

    @pl.when(t == 0)
    def _():
        y_ref[...] = z

    @pl.when(t != 0)
    def _():
        y_ref[...] += z

    @pl.when(t == n_grp - 1)
    def _():
        m = x[:, mcol:mcol + 1]
        xm = y_ref[...] * m
        c = jnp.sum(m) + 1e-8
        s1 = jnp.sum(xm, axis=0, keepdims=True)
        s2 = jnp.sum(xm * xm, axis=0, keepdims=True)
        mean = s1 / c
        var = s2 / c - mean * mean
        inv = jax.lax.rsqrt(var + _EPS) * g_ref[...]
        y = jnp.maximum((xm - mean) * inv + b_ref[...], 0.0) * m
        y_ref[...] = y


def _conv_split_kernel(x_ref, w_ref, xm_ref, s1_ref, s2_ref, cnt_ref, *,
                       n_grp, mcol):
    tg = pl.program_id(1)
    rb = pl.program_id(0)
    x = x_ref[0]
    z = jnp.dot(x, w_ref[0], preferred_element_type=_F32)

    @pl.when(tg == 0)
    def _():
        xm_ref[:, :z.shape[1]] = z

    @pl.when(tg != 0)
    def _():
        xm_ref[:, :z.shape[1]] += z

    @pl.when(tg == n_grp - 1)
    def _():
        m = x[:, mcol:mcol + 1]
        xm = xm_ref[:, :z.shape[1]] * m
        xm_ref[:, :z.shape[1]] = xm
        xm_ref[:, z.shape[1]:z.shape[1] + 1] = m
        s1 = jnp.sum(xm, axis=0, keepdims=True)
        s2 = jnp.sum(xm * xm, axis=0, keepdims=True)
        c = jnp.sum(m)[None, None]

        @pl.when(rb == 0)
        def _():
            s1_ref[...] = s1
            s2_ref[...] = s2
            cnt_ref[...] = c

        @pl.when(rb != 0)
        def _():
            s1_ref[...] += s1
            s2_ref[...] += s2
            cnt_ref[...] += c


def _norm_split_kernel(x_ref, g_ref, b_ref, s1_ref, s2_ref, cnt_ref, y_ref):
    c = cnt_ref[0, 0] + 1e-8
    mean = s1_ref[...] / c
    var = s2_ref[...] / c - mean * mean
    inv = jax.lax.rsqrt(var + _EPS) * g_ref[...]
    x = x_ref[...]
    cout = y_ref.shape[1]
    m = x[:, cout:cout + 1]
    y = (x[:, :cout] - mean) * inv + b_ref[...]
    y_ref[...] = jnp.maximum(y, 0.0) * m


def _conv_layer_split(taps, w, g, b, cout, mcol, n_rb):
    n_grp, r, k = taps.shape
    rblk = r // n_rb
    xm, s1, s2, cnt = pl.pallas_call(
        functools.partial(_conv_split_kernel, n_grp=n_grp, mcol=mcol),
        grid=(n_rb, n_grp),
        in_specs=[
            pl.BlockSpec((1, rblk, k), lambda rb, t: (t, rb, 0)),
            pl.BlockSpec((1, k, cout), lambda rb, t: (t, 0, 0)),
        ],
        out_specs=[
            pl.BlockSpec((rblk, cout + 1), lambda rb, t: (rb, 0)),
            pl.BlockSpec((1, cout), lambda rb, t: (0, 0)),
            pl.BlockSpec((1, cout), lambda rb, t: (0, 0)),
            pl.BlockSpec((1, 1), lambda rb, t: (0, 0)),
        ],
        out_shape=[
            jax.ShapeDtypeStruct((r, cout + 1), _F32),
            jax.ShapeDtypeStruct((1, cout), _F32),
            jax.ShapeDtypeStruct((1, cout), _F32),
            jax.ShapeDtypeStruct((1, 1), _F32),
        ],
        compiler_params=pltpu.CompilerParams(
            vmem_limit_bytes=64 * 1024 * 1024),
    )(taps, w)
    return pl.pallas_call(
        _norm_split_kernel,
        grid=(n_rb,),
        in_specs=[
            pl.BlockSpec((rblk, cout + 1), lambda rb: (rb, 0)),
            pl.BlockSpec((1, cout), lambda rb: (0, 0)),
            pl.BlockSpec((1, cout), lambda rb: (0, 0)),
            pl.BlockSpec((1, cout), lambda rb: (0, 0)),
            pl.BlockSpec((1, cout), lambda rb: (0, 0)),
            pl.BlockSpec((1, 1), lambda rb: (0, 0)),
        ],
        out_specs=pl.BlockSpec((rblk, cout), lambda rb: (rb, 0)),
        out_shape=jax.ShapeDtypeStruct((r, cout), _F32),
    )(xm, g, b, s1, s2, cnt)


def _conv_bn_dec_kernel(x_ref, w_ref, g_ref, b_ref, dw1_ref, db1_ref,
                        dw2_ref, db2_ref, p_ref, acc_ref, *, n_grp, mcol):
    t = pl.program_id(0)
    x = x_ref[0]
    z = jnp.dot(x, w_ref[0], preferred_element_type=_F32)

    @pl.when(t == 0)
    def _():
        acc_ref[...] = z

    @pl.when(t != 0)
    def _():
        acc_ref[...] += z

    @pl.when(t == n_grp - 1)
    def _():
        m = x[:, mcol:mcol + 1]
        xm = acc_ref[...] * m
        c = jnp.sum(m) + 1e-8
        s1 = jnp.sum(xm, axis=0, keepdims=True)
        s2 = jnp.sum(xm * xm, axis=0, keepdims=True)
        mean = s1 / c
        var = s2 / c - mean * mean
        inv = jax.lax.rsqrt(var + _EPS) * g_ref[...]
        y = jnp.maximum((xm - mean) * inv + b_ref[...], 0.0) * m
        h = jnp.maximum(
            jnp.dot(y, dw1_ref[...], preferred_element_type=_F32)
            + db1_ref[...], 0.0)
        p = jnp.dot(h, dw2_ref[...], preferred_element_type=_F32) + db2_ref[...]
        p_ref[...] = p * m


def _pool2(m):
    s = m.shape[1] // 2
    return m.reshape(m.shape[0], s, 2, s, 2, s, 2).max(axis=(2, 4, 6))


def _taps_grouped(y, cin, mrow, tpg):
    B, g = y.shape[0], y.shape[1]
    o1 = g // 2 + 1
    o = o1 - 1
    r = B * o ** 3
    yp = jnp.pad(y, ((0, 0), (1, 1), (1, 1), (1, 1), (0, 0)))
    y8 = yp.reshape(B, o1, 2, o1, 2, o1, 2 * cin)
    cols = []
    for a in range(3):
        for b in range(3):
            for c in range(3):
                pd, sd = a % 2, a // 2
                ph, sh = b % 2, b // 2
                pw, sw = c % 2, c // 2
                sl = y8[:, sd:sd + o, pd, sh:sh + o, ph, sw:sw + o,
                        pw * cin:(pw + 1) * cin]
                cols.append(sl.reshape(r, cin))
    rider = jnp.concatenate([mrow, jnp.zeros((r, cin - 1), _F32)], axis=1)
    cols.append(rider)
    n_grp = -(-len(cols) // tpg)
    while len(cols) < n_grp * tpg:
        cols.append(jnp.zeros((r, cin), _F32))
    groups = [jnp.concatenate(cols[i * tpg:(i + 1) * tpg], axis=1)
              for i in range(n_grp)]
    return jnp.stack(groups, axis=0)


def _group_weights(w, cout, tpg):
    n_taps, cin, _ = w.shape
    n_grp = -(-(n_taps + 1) // tpg)
    wp = jnp.concatenate(
        [w, jnp.zeros((n_grp * tpg - n_taps, cin, cout), _F32)], axis=0)
    return wp.reshape(n_grp, tpg * cin, cout)


def _conv_layer(taps, w, g, b, cout, mcol):
    n_grp, r, k = taps.shape
    return pl.pallas_call(
        functools.partial(_conv_bn_kernel, n_grp=n_grp, mcol=mcol),
        grid=(n_grp,),
        in_specs=[
            pl.BlockSpec((1, r, k), lambda t: (t, 0, 0)),
            pl.BlockSpec((1, k, cout), lambda t: (t, 0, 0)),
            pl.BlockSpec((1, cout), lambda t: (0, 0)),
            pl.BlockSpec((1, cout), lambda t: (0, 0)),
        ],
        out_specs=pl.BlockSpec((r, cout), lambda t: (0, 0)),
        out_shape=jax.ShapeDtypeStruct((r, cout), _F32),
        compiler_params=pltpu.CompilerParams(
            vmem_limit_bytes=64 * 1024 * 1024),
    )(taps, w, g, b)


def _conv_layer_dec(taps, w, g, b, dw1, db1, dw2, db2, cout, mcol):
    n_grp, r, k = taps.shape
    hid = dw1.shape[1]
    return pl.pallas_call(
        functools.partial(_conv_bn_dec_kernel, n_grp=n_grp, mcol=mcol),
        grid=(n_grp,),
        in_specs=[
            pl.BlockSpec((1, r, k), lambda t: (t, 0, 0)),
            pl.BlockSpec((1, k, cout), lambda t: (t, 0, 0)),
            pl.BlockSpec((1, cout), lambda t: (0, 0)),
            pl.BlockSpec((1, cout), lambda t: (0, 0)),
            pl.BlockSpec((cout, hid), lambda t: (0, 0)),
            pl.BlockSpec((1, hid), lambda t: (0, 0)),
            pl.BlockSpec((hid, 1), lambda t: (0, 0)),
            pl.BlockSpec((1, 1), lambda t: (0, 0)),
        ],
        out_specs=pl.BlockSpec((r, 1), lambda t: (0, 0)),
        out_shape=jax.ShapeDtypeStruct((r, 1), _F32),
        scratch_shapes=[pltpu.VMEM((r, cout), _F32)],
        compiler_params=pltpu.CompilerParams(
            vmem_limit_bytes=64 * 1024 * 1024),
    )(taps, w, g, b, dw1, db1, dw2, db2)


def kernel(images, mask, conv_w0, bn_g0, bn_b0, conv_w1, bn_g1, bn_b1,
           conv_w2, bn_g2, bn_b2, conv_w3, bn_g3, bn_b3,
           dec_w1, dec_b1, dec_w2, dec_b2):
    B, _, D = images.shape[0], images.shape[1], images.shape[2]
    m0 = mask.astype(_F32)[:, 0]
    x = images[:, 0] * m0

    m1 = _pool2(m0)
    m2 = _pool2(m1)
    m3 = _pool2(m2)

    r0 = B * D ** 3
    rg = B * D * D * (D // 4)
    xp4 = jnp.pad(x, ((0, 0), (1, 1), (1, 1), (1, 3))).reshape(
        B, D + 2, D + 2, (D + 4) // 4, 4)
    cols = [
        xp4[:, a:a + D, b:b + D, c // 4:c // 4 + D // 4, c % 4]
        for a in range(3) for b in range(3) for c in range(6)
    ]
    x0 = jnp.concatenate([
        jnp.stack(cols, axis=-1).reshape(rg, 54),
        m0.reshape(rg, 4),
        jnp.zeros((rg, 70), _F32),
    ], axis=1)
    w0b = jnp.pad(conv_w0.reshape(9, 3, 64), ((0, 0), (0, 3), (0, 0)))
    w0b = w0b.reshape(54, 64)
    w0g = jnp.concatenate(
        [jnp.pad(w0b, ((j, 74 - j), (0, 0))) for j in range(4)], axis=1)
    blk = rg // 4
    s1, s2, cnt = pl.pallas_call(
        _l0_stats_kernel,
        grid=(4,),
        in_specs=[
            pl.BlockSpec((blk, 128), lambda i: (i, 0)),
            pl.BlockSpec((128, 256), lambda i: (0, 0)),
        ],
        out_specs=[
            pl.BlockSpec((1, 256), lambda i: (0, 0)),
            pl.BlockSpec((1, 256), lambda i: (0, 0)),
            pl.BlockSpec((1, 1), lambda i: (0, 0)),
        ],
        out_shape=[
            jax.ShapeDtypeStruct((1, 256), _F32),
            jax.ShapeDtypeStruct((1, 256), _F32),
            jax.ShapeDtypeStruct((1, 1), _F32),
        ],
    )(x0, w0g)

    y0 = pl.pallas_call(
        _l0_norm_kernel,
        grid=(4,),
        in_specs=[
            pl.BlockSpec((blk, 128), lambda i: (i, 0)),
            pl.BlockSpec((128, 256), lambda i: (0, 0)),
            pl.BlockSpec((1, 64), lambda i: (0, 0)),
            pl.BlockSpec((1, 64), lambda i: (0, 0)),
            pl.BlockSpec((1, 256), lambda i: (0, 0)),
            pl.BlockSpec((1, 256), lambda i: (0, 0)),
            pl.BlockSpec((1, 1), lambda i: (0, 0)),
        ],
        out_specs=pl.BlockSpec((blk, 256), lambda i: (i, 0)),
        out_shape=jax.ShapeDtypeStruct((rg, 256), _F32),
    )(x0, w0g, bn_g0.reshape(1, 64), bn_b0.reshape(1, 64), s1, s2, cnt)
    y0 = y0.reshape(r0, 64)

    y = y0
    g = D
    chans = [64, 128, 256]
    convs = [(conv_w1, bn_g1, bn_b1), (conv_w2, bn_g2, bn_b2)]
    masks = [m1, m2]
    for i, (cw, bg, bb) in enumerate(convs):
        cin, cout = chans[i], chans[i + 1]
        tpg = 256 // cin
        r = B * (g // 2) ** 3
        taps = _taps_grouped(y.reshape(B, g, g, g, cin), cin,
                             masks[i].reshape(r, 1), tpg)
        g = g // 2
        gw = _group_weights(cw.reshape(27, cin, cout), cout, tpg)
        mcol = 27 % tpg * cin
        if i == 0:
            y = _conv_layer_split(taps, gw, bg.reshape(1, cout),
                                  bb.reshape(1, cout), cout, mcol, n_rb=2)
        else:
            y = _conv_layer(taps, gw, bg.reshape(1, cout),
                            bb.reshape(1, cout), cout, mcol)

    cin, cout = 256, 768
    tpg = 1
    r = B * (g // 2) ** 3
    taps = _taps_grouped(y.reshape(B, g, g, g, cin), cin, m3.reshape(r, 1),
                         tpg)
    g = g // 2
    p = _conv_layer_dec(taps, _group_weights(conv_w3.reshape(27, cin, cout),
                                             cout, tpg),
                        bn_g3.reshape(1, cout), bn_b3.reshape(1, cout),
                        dec_w1.T, dec_b1.reshape(1, -1), dec_w2.T,
                        dec_b2.reshape(1, 1), cout, mcol=0)

    recon = jnp.zeros((B, 1, D, D, D), _F32)
    recon = recon.at[:, :, :g, :g, :g].set(p.reshape(B, 1, g, g, g))
    return recon
